# baseline (speedup 1.0000x reference)
import dataclasses
import functools

import jax
import jax.numpy as jnp
from jax import lax
from jax.experimental import pallas as pl
from jax.experimental.pallas import tpu as pltpu
from jax.experimental.pallas import tpu_sc as plsc

N = 10000
E = 640000
IN_CH = 128
HID = 16
HEADS = 5
OUT_CH = 16

NP = 10240
CHUNK = 128
NCHUNKS = E // CHUNK
NW = 32
ROWS_PER_TILE = NP // 16
ACC_W = 96


def _sc_compiler_params():
    cp = pltpu.CompilerParams(use_tc_tiling_on_sc=False)
    if "needs_layout_passes" in pltpu.CompilerParams.__dataclass_fields__:
        cp = dataclasses.replace(cp, needs_layout_passes=False)
    return cp


def _dg(v, idx):
    dnums = lax.GatherDimensionNumbers(
        offset_dims=(), collapsed_slice_dims=(0,), start_index_map=(0,))
    return lax.gather(v, idx[:, None], dnums, slice_sizes=(1,),
                      mode=lax.GatherScatterMode.PROMISE_IN_BOUNDS)


NCH_W = 160
ECHUNKS_PAD = NW * NCH_W
IBLK = 16
NBLK = NCH_W // IBLK


def _edge_pass(src2d, dst2d, h, dp):
    mesh = plsc.VectorSubcoreMesh(core_axis_name="c", subcore_axis_name="s")

    @functools.partial(
        pl.kernel,
        mesh=mesh,
        out_type=jax.ShapeDtypeStruct((2, NP, ACC_W), jnp.float32),
        scratch_types=[
            pltpu.VMEM((IBLK, CHUNK), jnp.int32),
            pltpu.VMEM((IBLK, CHUNK), jnp.int32),
            pltpu.VMEM((CHUNK, 16), jnp.float32),
            pltpu.VMEM((CHUNK, 16), jnp.float32),
            pltpu.VMEM((CHUNK, ACC_W), jnp.float32),
            pltpu.VMEM((CHUNK, ACC_W), jnp.float32),
            pltpu.VMEM((CHUNK, ACC_W), jnp.float32),
            pltpu.VMEM((CHUNK, ACC_W), jnp.float32),
            pltpu.VMEM_SHARED((NP, ACC_W), jnp.float32),
            pltpu.SemaphoreType.DMA,
            pltpu.SemaphoreType.DMA,
            pltpu.SemaphoreType.DMA,
            pltpu.SemaphoreType.DMA,
        ],
        compiler_params=_sc_compiler_params(),
    )
    def k(src_hbm, dst_hbm, h_hbm, dp_hbm, out_hbm,
          sidx, didx, dbuf0, dbuf1, hbuf0, hbuf1,
          obuf0, obuf1, acc, gsem0, gsem1, ssem0, ssem1):
        cid = lax.axis_index("c")
        sid = lax.axis_index("s")
        wid = sid * 2 + cid
        c0 = wid * NCH_W

        dbuf = (dbuf0, dbuf1)
        hbuf = (hbuf0, hbuf1)
        obuf = (obuf0, obuf1)
        gsem = (gsem0, gsem1)
        ssem = (ssem0, ssem1)

        zv = jnp.zeros((16,), jnp.float32)

        @pl.loop(0, CHUNK)
        def _(r):
            for c in range(ACC_W // 16):
                obuf0[r, pl.ds(c * 16, 16)] = zv

        for b in range(ROWS_PER_TILE // CHUNK):
            pltpu.sync_copy(obuf0, acc.at[pl.ds(sid * ROWS_PER_TILE + b * CHUNK, CHUNK)])
        plsc.subcore_barrier()

        def refill_idx(blk):
            pltpu.sync_copy(src_hbm.at[pl.ds(c0 + blk * IBLK, IBLK)], sidx)
            pltpu.sync_copy(dst_hbm.at[pl.ds(c0 + blk * IBLK, IBLK)], didx)

        def issue(row, b):
            pltpu.async_copy(dp_hbm.at[didx.at[row]], dbuf[b], gsem[b])
            pltpu.async_copy(h_hbm.at[sidx.at[row]], hbuf[b], gsem[b])

        def drain(b):
            pltpu.make_async_copy(dp_hbm.at[didx.at[0]], dbuf[b], gsem[b]).wait()
            pltpu.make_async_copy(h_hbm.at[sidx.at[0]], hbuf[b], gsem[b]).wait()

        iota16 = lax.iota(jnp.int32, 16)
        shift8 = jnp.minimum(iota16 + 8, 15)
        head_idx = [jnp.full((16,), hh, jnp.int32) for hh in range(HEADS)]

        def compute(b, it, row):
            drain(b)
            ob = obuf[b]
            hb = hbuf[b]

            @pl.when(row >= 2)
            def _():
                pltpu.make_async_copy(ob, acc.at[didx.at[0]], ssem[b]).wait()

            @plsc.parallel_loop(0, CHUNK, unroll=4)
            def _(e):
                sreg = hb[e, pl.ds(80, 16)]
                dreg = dbuf[b][e]
                u = sreg + dreg
                l = jnp.maximum(u, 0.2 * u)
                ub = _dg(dreg, shift8)
                t = jnp.exp(l - ub)
                ob[e, pl.ds(80, 16)] = t
                for hh in range(HEADS):
                    cf = _dg(t, head_idx[hh])
                    ob[e, pl.ds(hh * 16, 16)] = hb[e, pl.ds(hh * 16, 16)] * cf

            pltpu.async_copy(ob, acc.at[didx.at[row]], ssem[b], add=True)

        refill_idx(0)
        issue(0, 0)

        @pl.loop(0, NBLK)
        def _(bb):
            @pl.loop(0, IBLK // 2)
            def _(cc):
                for b in range(2):
                    j = cc * 2 + b
                    it = bb * IBLK + j

                    if b == 0:
                        @pl.when(c0 + it + 1 < NCHUNKS)
                        def _():
                            issue(j + 1, 1 - b)

                        @pl.when(c0 + it < NCHUNKS)
                        def _():
                            compute(b, it, j)
                    else:
                        @pl.when((cc < IBLK // 2 - 1) & (c0 + it + 1 < NCHUNKS))
                        def _():
                            issue(j + 1, 1 - b)

                        @pl.when(c0 + it < NCHUNKS)
                        def _():
                            compute(b, it, j)

                        @pl.when((cc == IBLK // 2 - 1) & (bb + 1 < NBLK))
                        def _():
                            @pl.when(c0 + it - 1 < NCHUNKS)
                            def _():
                                pltpu.make_async_copy(
                                    obuf[0], acc.at[didx.at[0]], ssem[0]).wait()

                            @pl.when(c0 + it < NCHUNKS)
                            def _():
                                pltpu.make_async_copy(
                                    obuf[1], acc.at[didx.at[0]], ssem[1]).wait()

                            refill_idx(bb + 1)

                            @pl.when(c0 + it + 1 < NCHUNKS)
                            def _():
                                issue(0, 1 - b)

        pltpu.make_async_copy(obuf[0], acc.at[didx.at[0]], ssem[0]).wait()
        pltpu.make_async_copy(obuf[1], acc.at[didx.at[0]], ssem[1]).wait()
        plsc.subcore_barrier()
        pltpu.sync_copy(acc.at[pl.ds(sid * ROWS_PER_TILE, ROWS_PER_TILE)],
                        out_hbm.at[cid, pl.ds(sid * ROWS_PER_TILE, ROWS_PER_TILE)])

    return k(src2d, dst2d, h, dp)


def _head_sum_mat(att_flat):
    r = lax.broadcasted_iota(jnp.int32, (80, HEADS), 0)
    c = lax.broadcasted_iota(jnp.int32, (80, HEADS), 1)
    sel = (r // 16 == c).astype(jnp.float32)
    return att_flat[:, None] * sel


def _expand_mat():
    r = lax.broadcasted_iota(jnp.int32, (HEADS, 80), 0)
    c = lax.broadcasted_iota(jnp.int32, (HEADS, 80), 1)
    return (c // 16 == r).astype(jnp.float32)


def _attn_packs(hmat, att_s_flat, att_d_flat):
    a_s = jnp.dot(hmat, _head_sum_mat(att_s_flat),
                  preferred_element_type=jnp.float32)
    a_d = jnp.dot(hmat, _head_sum_mat(att_d_flat),
                  preferred_element_type=jnp.float32)
    gmax = jnp.max(a_s, axis=0, keepdims=True)
    v = gmax + a_d
    ub = jnp.maximum(v, 0.2 * v)
    z = jnp.zeros((hmat.shape[0], 3), jnp.float32)
    sp = jnp.concatenate([a_s, z, a_s, z], axis=1)
    dp = jnp.concatenate([a_d, z, ub, z], axis=1)
    return sp, dp


def _pre1_body(x_ref, wemb_ref, bemb_ref, w1_ref, as1_ref, ad1_ref,
               emb_ref, h_ref, dp_ref):
    emb = jnp.dot(x_ref[...], wemb_ref[...],
                  preferred_element_type=jnp.float32) + bemb_ref[...]
    emb_ref[...] = emb
    h = jnp.dot(emb, w1_ref[...], preferred_element_type=jnp.float32)
    sp, dp = _attn_packs(h, as1_ref[...][0], ad1_ref[...][0])
    h_ref[...] = jnp.concatenate([h, sp], axis=1)
    dp_ref[...] = dp


def _mid_body(acc_ref, b1_ref, w2_ref, as2_ref, ad2_ref,
              h_ref, dp_ref):
    s = acc_ref[0] + acc_ref[1]
    msg = s[:, :80]
    den = jnp.dot(s[:, 80:85], _expand_mat(),
                  preferred_element_type=jnp.float32)
    o = msg / (den + 1e-16) + b1_ref[...]
    x2 = jnp.where(o > 0, o, jnp.exp(jnp.minimum(o, 0.0)) - 1.0)
    h = jnp.dot(x2, w2_ref[...], preferred_element_type=jnp.float32)
    sp, dp = _attn_packs(h, as2_ref[...][0], ad2_ref[...][0])
    h_ref[...] = jnp.concatenate([h, sp], axis=1)
    dp_ref[...] = dp


def _post_body(acc_ref, b2_ref, out_ref):
    s = acc_ref[0] + acc_ref[1]
    msg = s[:, :80]
    den = jnp.dot(s[:, 80:85], _expand_mat(),
                  preferred_element_type=jnp.float32)
    o = msg / (den + 1e-16)
    r = lax.broadcasted_iota(jnp.int32, (80, OUT_CH), 0)
    c = lax.broadcasted_iota(jnp.int32, (80, OUT_CH), 1)
    mh = (r % 16 == c).astype(jnp.float32) / HEADS
    om = jnp.dot(o, mh, preferred_element_type=jnp.float32) + b2_ref[...]
    m = jnp.max(om, axis=1, keepdims=True)
    z = om - m
    lse = jnp.log(jnp.sum(jnp.exp(z), axis=1, keepdims=True))
    out_ref[...] = z - lse


def kernel(x, edge_index, W_emb, b_emb, W1, att_src1, att_dst1, b1,
           W2, att_src2, att_dst2, b2):
    pad_e = ECHUNKS_PAD * CHUNK - E
    src = jnp.concatenate([edge_index[0], jnp.zeros((pad_e,), jnp.int32)])
    src2d = src.reshape(ECHUNKS_PAD, CHUNK)
    dst = jnp.concatenate([edge_index[1], jnp.zeros((pad_e,), jnp.int32)])
    dst2d = dst.reshape(ECHUNKS_PAD, CHUNK)
    as1 = att_src1.reshape(1, HEADS * HID)
    ad1 = att_dst1.reshape(1, HEADS * HID)
    as2 = att_src2.reshape(1, HEADS * OUT_CH)
    ad2 = att_dst2.reshape(1, HEADS * OUT_CH)
    b_emb2 = b_emb.reshape(1, HID)
    b1r = b1.reshape(1, HEADS * HID)
    b2r = b2.reshape(1, OUT_CH)
    x_pad = jnp.concatenate(
        [x, jnp.zeros((NP - N, IN_CH), jnp.float32)], axis=0)

    emb, h1, dp1 = pl.pallas_call(
        _pre1_body,
        out_shape=[
            jax.ShapeDtypeStruct((NP, HID), jnp.float32),
            jax.ShapeDtypeStruct((NP, 96), jnp.float32),
            jax.ShapeDtypeStruct((NP, 16), jnp.float32),
        ],
    )(x_pad, W_emb, b_emb2, W1, as1, ad1)

    acc1 = _edge_pass(src2d, dst2d, h1, dp1)

    h2, dp2 = pl.pallas_call(
        _mid_body,
        out_shape=[
            jax.ShapeDtypeStruct((NP, 96), jnp.float32),
            jax.ShapeDtypeStruct((NP, 16), jnp.float32),
        ],
    )(acc1, b1r, W2, as2, ad2)

    acc2 = _edge_pass(src2d, dst2d, h2, dp2)

    out = pl.pallas_call(
        _post_body,
        out_shape=jax.ShapeDtypeStruct((NP, OUT_CH), jnp.float32),
    )(acc2, b2r)

    return (emb[:N], out[:N])


# speedup vs baseline: 1.1059x; 1.1059x over previous
import dataclasses
import functools

import jax
import jax.numpy as jnp
from jax import lax
from jax.experimental import pallas as pl
from jax.experimental.pallas import tpu as pltpu
from jax.experimental.pallas import tpu_sc as plsc

N = 10000
E = 640000
IN_CH = 128
HID = 16
HEADS = 5
OUT_CH = 16

NP = 10240
CHUNK = 128
NCHUNKS = E // CHUNK
NW = 32
ROWS_PER_TILE = NP // 16
ACC_W = 96


def _sc_compiler_params():
    cp = pltpu.CompilerParams(use_tc_tiling_on_sc=False)
    if "needs_layout_passes" in pltpu.CompilerParams.__dataclass_fields__:
        cp = dataclasses.replace(cp, needs_layout_passes=False)
    return cp


def _dg(v, idx):
    dnums = lax.GatherDimensionNumbers(
        offset_dims=(), collapsed_slice_dims=(0,), start_index_map=(0,))
    return lax.gather(v, idx[:, None], dnums, slice_sizes=(1,),
                      mode=lax.GatherScatterMode.PROMISE_IN_BOUNDS)


NCH_W = 160
ECHUNKS_PAD = NW * NCH_W
IBLK = 16
NBLK = NCH_W // IBLK


def _edge_pass(src2d, dst2d, h, dp):
    mesh = plsc.VectorSubcoreMesh(core_axis_name="c", subcore_axis_name="s")

    @functools.partial(
        pl.kernel,
        mesh=mesh,
        out_type=jax.ShapeDtypeStruct((2, NP, ACC_W), jnp.float32),
        scratch_types=[
            pltpu.VMEM((IBLK, CHUNK), jnp.int32),
            pltpu.VMEM((IBLK, CHUNK), jnp.int32),
            pltpu.VMEM((CHUNK, 16), jnp.float32),
            pltpu.VMEM((CHUNK, 16), jnp.float32),
            pltpu.VMEM((CHUNK, ACC_W), jnp.bfloat16),
            pltpu.VMEM((CHUNK, ACC_W), jnp.bfloat16),
            pltpu.VMEM((CHUNK, ACC_W), jnp.float32),
            pltpu.VMEM((CHUNK, ACC_W), jnp.float32),
            pltpu.VMEM_SHARED((NP, ACC_W), jnp.float32),
            pltpu.SemaphoreType.DMA,
            pltpu.SemaphoreType.DMA,
            pltpu.SemaphoreType.DMA,
            pltpu.SemaphoreType.DMA,
        ],
        compiler_params=_sc_compiler_params(),
    )
    def k(src_hbm, dst_hbm, h_hbm, dp_hbm, out_hbm,
          sidx, didx, dbuf0, dbuf1, hbuf0, hbuf1,
          obuf0, obuf1, acc, gsem0, gsem1, ssem0, ssem1):
        cid = lax.axis_index("c")
        sid = lax.axis_index("s")
        wid = sid * 2 + cid
        c0 = wid * NCH_W

        dbuf = (dbuf0, dbuf1)
        hbuf = (hbuf0, hbuf1)
        obuf = (obuf0, obuf1)
        gsem = (gsem0, gsem1)
        ssem = (ssem0, ssem1)

        zv = jnp.zeros((16,), jnp.float32)

        @pl.loop(0, CHUNK)
        def _(r):
            for c in range(ACC_W // 16):
                obuf0[r, pl.ds(c * 16, 16)] = zv

        for b in range(ROWS_PER_TILE // CHUNK):
            pltpu.sync_copy(obuf0, acc.at[pl.ds(sid * ROWS_PER_TILE + b * CHUNK, CHUNK)])
        plsc.subcore_barrier()

        def refill_idx(blk):
            pltpu.sync_copy(src_hbm.at[pl.ds(c0 + blk * IBLK, IBLK)], sidx)
            pltpu.sync_copy(dst_hbm.at[pl.ds(c0 + blk * IBLK, IBLK)], didx)

        def issue(row, b):
            pltpu.async_copy(dp_hbm.at[didx.at[row]], dbuf[b], gsem[b])
            pltpu.async_copy(h_hbm.at[sidx.at[row]], hbuf[b], gsem[b])

        def drain(b):
            pltpu.make_async_copy(dp_hbm.at[didx.at[0]], dbuf[b], gsem[b]).wait()
            pltpu.make_async_copy(h_hbm.at[sidx.at[0]], hbuf[b], gsem[b]).wait()

        iota16 = lax.iota(jnp.int32, 16)
        shift8 = jnp.minimum(iota16 + 8, 15)
        head_idx = [jnp.full((16,), hh, jnp.int32) for hh in range(HEADS)]

        def compute(b, it, row):
            drain(b)
            ob = obuf[b]
            hb = hbuf[b]

            @pl.when(row >= 2)
            def _():
                pltpu.make_async_copy(ob, acc.at[didx.at[0]], ssem[b]).wait()

            @plsc.parallel_loop(0, CHUNK, unroll=4)
            def _(e):
                r2 = hb[e, pl.ds(64, 32)]
                h4, sreg = plsc.unpack(
                    r2, format=plsc.PackFormat.INTERLEAVED,
                    preferred_element_type=jnp.float32)
                dreg = dbuf[b][e]
                u = sreg + dreg
                l = jnp.maximum(u, 0.2 * u)
                ub = _dg(dreg, shift8)
                t = jnp.exp(l - ub)
                ob[e, pl.ds(80, 16)] = t
                r0 = hb[e, pl.ds(0, 32)]
                h0, h1 = plsc.unpack(
                    r0, format=plsc.PackFormat.INTERLEAVED,
                    preferred_element_type=jnp.float32)
                r1 = hb[e, pl.ds(32, 32)]
                h2, h3 = plsc.unpack(
                    r1, format=plsc.PackFormat.INTERLEAVED,
                    preferred_element_type=jnp.float32)
                hs = (h0, h1, h2, h3, h4)
                for hh in range(HEADS):
                    cf = _dg(t, head_idx[hh])
                    ob[e, pl.ds(hh * 16, 16)] = hs[hh] * cf

            pltpu.async_copy(ob, acc.at[didx.at[row]], ssem[b], add=True)

        refill_idx(0)
        issue(0, 0)

        @pl.loop(0, NBLK)
        def _(bb):
            @pl.loop(0, IBLK // 2)
            def _(cc):
                for b in range(2):
                    j = cc * 2 + b
                    it = bb * IBLK + j

                    if b == 0:
                        @pl.when(c0 + it + 1 < NCHUNKS)
                        def _():
                            issue(j + 1, 1 - b)

                        @pl.when(c0 + it < NCHUNKS)
                        def _():
                            compute(b, it, j)
                    else:
                        @pl.when((cc < IBLK // 2 - 1) & (c0 + it + 1 < NCHUNKS))
                        def _():
                            issue(j + 1, 1 - b)

                        @pl.when(c0 + it < NCHUNKS)
                        def _():
                            compute(b, it, j)

                        @pl.when((cc == IBLK // 2 - 1) & (bb + 1 < NBLK))
                        def _():
                            @pl.when(c0 + it - 1 < NCHUNKS)
                            def _():
                                pltpu.make_async_copy(
                                    obuf[0], acc.at[didx.at[0]], ssem[0]).wait()

                            @pl.when(c0 + it < NCHUNKS)
                            def _():
                                pltpu.make_async_copy(
                                    obuf[1], acc.at[didx.at[0]], ssem[1]).wait()

                            refill_idx(bb + 1)

                            @pl.when(c0 + it + 1 < NCHUNKS)
                            def _():
                                issue(0, 1 - b)

        pltpu.make_async_copy(obuf[0], acc.at[didx.at[0]], ssem[0]).wait()
        pltpu.make_async_copy(obuf[1], acc.at[didx.at[0]], ssem[1]).wait()
        plsc.subcore_barrier()
        pltpu.sync_copy(acc.at[pl.ds(sid * ROWS_PER_TILE, ROWS_PER_TILE)],
                        out_hbm.at[cid, pl.ds(sid * ROWS_PER_TILE, ROWS_PER_TILE)])

    return k(src2d, dst2d, h, dp)


def _head_sum_mat(att_flat):
    r = lax.broadcasted_iota(jnp.int32, (80, HEADS), 0)
    c = lax.broadcasted_iota(jnp.int32, (80, HEADS), 1)
    sel = (r // 16 == c).astype(jnp.float32)
    return att_flat[:, None] * sel


def _expand_mat():
    r = lax.broadcasted_iota(jnp.int32, (HEADS, 80), 0)
    c = lax.broadcasted_iota(jnp.int32, (HEADS, 80), 1)
    return (c // 16 == r).astype(jnp.float32)


def _perm_mat():
    lrow = lax.broadcasted_iota(jnp.int32, (96, 96), 0)
    s = lax.broadcasted_iota(jnp.int32, (96, 96), 1)
    q = s // 32
    r = s % 32
    l_of_s = 32 * q + r // 2 + 16 * (r % 2)
    return (lrow == l_of_s).astype(jnp.float32)


def _attn_packs(hmat, att_s_flat, att_d_flat):
    a_s = jnp.dot(hmat, _head_sum_mat(att_s_flat),
                  preferred_element_type=jnp.float32)
    a_d = jnp.dot(hmat, _head_sum_mat(att_d_flat),
                  preferred_element_type=jnp.float32)
    gmax = jnp.max(a_s, axis=0, keepdims=True)
    v = gmax + a_d
    ub = jnp.maximum(v, 0.2 * v)
    z = jnp.zeros((hmat.shape[0], 3), jnp.float32)
    sp = jnp.concatenate([a_s, z, a_s, z], axis=1)
    dp = jnp.concatenate([a_d, z, ub, z], axis=1)
    return sp, dp


def _pre1_body(x_ref, wemb_ref, bemb_ref, w1_ref, as1_ref, ad1_ref,
               emb_ref, h_ref, dp_ref):
    emb = jnp.dot(x_ref[...], wemb_ref[...],
                  preferred_element_type=jnp.float32) + bemb_ref[...]
    emb_ref[...] = emb
    h = jnp.dot(emb, w1_ref[...], preferred_element_type=jnp.float32)
    sp, dp = _attn_packs(h, as1_ref[...][0], ad1_ref[...][0])
    h96 = jnp.concatenate([h, sp], axis=1)
    h_ref[...] = jnp.dot(h96, _perm_mat(),
                         preferred_element_type=jnp.float32).astype(jnp.bfloat16)
    dp_ref[...] = dp


def _mid_body(acc_ref, b1_ref, w2_ref, as2_ref, ad2_ref,
              h_ref, dp_ref):
    s = acc_ref[0] + acc_ref[1]
    msg = s[:, :80]
    den = jnp.dot(s[:, 80:85], _expand_mat(),
                  preferred_element_type=jnp.float32)
    o = msg / (den + 1e-16) + b1_ref[...]
    x2 = jnp.where(o > 0, o, jnp.exp(jnp.minimum(o, 0.0)) - 1.0)
    h = jnp.dot(x2, w2_ref[...], preferred_element_type=jnp.float32)
    sp, dp = _attn_packs(h, as2_ref[...][0], ad2_ref[...][0])
    h96 = jnp.concatenate([h, sp], axis=1)
    h_ref[...] = jnp.dot(h96, _perm_mat(),
                         preferred_element_type=jnp.float32).astype(jnp.bfloat16)
    dp_ref[...] = dp


def _post_body(acc_ref, b2_ref, out_ref):
    s = acc_ref[0] + acc_ref[1]
    msg = s[:, :80]
    den = jnp.dot(s[:, 80:85], _expand_mat(),
                  preferred_element_type=jnp.float32)
    o = msg / (den + 1e-16)
    r = lax.broadcasted_iota(jnp.int32, (80, OUT_CH), 0)
    c = lax.broadcasted_iota(jnp.int32, (80, OUT_CH), 1)
    mh = (r % 16 == c).astype(jnp.float32) / HEADS
    om = jnp.dot(o, mh, preferred_element_type=jnp.float32) + b2_ref[...]
    m = jnp.max(om, axis=1, keepdims=True)
    z = om - m
    lse = jnp.log(jnp.sum(jnp.exp(z), axis=1, keepdims=True))
    out_ref[...] = z - lse


def kernel(x, edge_index, W_emb, b_emb, W1, att_src1, att_dst1, b1,
           W2, att_src2, att_dst2, b2):
    pad_e = ECHUNKS_PAD * CHUNK - E
    src = jnp.concatenate([edge_index[0], jnp.zeros((pad_e,), jnp.int32)])
    src2d = src.reshape(ECHUNKS_PAD, CHUNK)
    dst = jnp.concatenate([edge_index[1], jnp.zeros((pad_e,), jnp.int32)])
    dst2d = dst.reshape(ECHUNKS_PAD, CHUNK)
    as1 = att_src1.reshape(1, HEADS * HID)
    ad1 = att_dst1.reshape(1, HEADS * HID)
    as2 = att_src2.reshape(1, HEADS * OUT_CH)
    ad2 = att_dst2.reshape(1, HEADS * OUT_CH)
    b_emb2 = b_emb.reshape(1, HID)
    b1r = b1.reshape(1, HEADS * HID)
    b2r = b2.reshape(1, OUT_CH)
    x_pad = jnp.concatenate(
        [x, jnp.zeros((NP - N, IN_CH), jnp.float32)], axis=0)

    emb, h1, dp1 = pl.pallas_call(
        _pre1_body,
        out_shape=[
            jax.ShapeDtypeStruct((NP, HID), jnp.float32),
            jax.ShapeDtypeStruct((NP, 96), jnp.bfloat16),
            jax.ShapeDtypeStruct((NP, 16), jnp.float32),
        ],
    )(x_pad, W_emb, b_emb2, W1, as1, ad1)

    acc1 = _edge_pass(src2d, dst2d, h1, dp1)

    h2, dp2 = pl.pallas_call(
        _mid_body,
        out_shape=[
            jax.ShapeDtypeStruct((NP, 96), jnp.bfloat16),
            jax.ShapeDtypeStruct((NP, 16), jnp.float32),
        ],
    )(acc1, b1r, W2, as2, ad2)

    acc2 = _edge_pass(src2d, dst2d, h2, dp2)

    out = pl.pallas_call(
        _post_body,
        out_shape=jax.ShapeDtypeStruct((NP, OUT_CH), jnp.float32),
    )(acc2, b2r)

    return (emb[:N], out[:N])


# speedup vs baseline: 1.1468x; 1.0369x over previous
import dataclasses
import functools

import jax
import jax.numpy as jnp
from jax import lax
from jax.experimental import pallas as pl
from jax.experimental.pallas import tpu as pltpu
from jax.experimental.pallas import tpu_sc as plsc

N = 10000
E = 640000
IN_CH = 128
HID = 16
HEADS = 5
OUT_CH = 16

NP = 10240
CHUNK = 128
NCHUNKS = E // CHUNK
NW = 32
ROWS_PER_TILE = NP // 16
ACC_W = 96


def _sc_compiler_params():
    cp = pltpu.CompilerParams(use_tc_tiling_on_sc=False)
    if "needs_layout_passes" in pltpu.CompilerParams.__dataclass_fields__:
        cp = dataclasses.replace(cp, needs_layout_passes=False)
    return cp


def _dg(v, idx):
    dnums = lax.GatherDimensionNumbers(
        offset_dims=(), collapsed_slice_dims=(0,), start_index_map=(0,))
    return lax.gather(v, idx[:, None], dnums, slice_sizes=(1,),
                      mode=lax.GatherScatterMode.PROMISE_IN_BOUNDS)


NCH_W = 160
ECHUNKS_PAD = NW * NCH_W
IBLK = 16
NBLK = NCH_W // IBLK


def _edge_pass(src2d, dst2d, h, dp):
    mesh = plsc.VectorSubcoreMesh(core_axis_name="c", subcore_axis_name="s")

    @functools.partial(
        pl.kernel,
        mesh=mesh,
        out_type=jax.ShapeDtypeStruct((2, NP, ACC_W), jnp.float32),
        scratch_types=[
            pltpu.VMEM((IBLK, CHUNK), jnp.int32),
            pltpu.VMEM((IBLK, CHUNK), jnp.int32),
            pltpu.VMEM((CHUNK, 16), jnp.float32),
            pltpu.VMEM((CHUNK, 16), jnp.float32),
            pltpu.VMEM((CHUNK, 16), jnp.float32),
            pltpu.VMEM((CHUNK, 16), jnp.float32),
            pltpu.VMEM((CHUNK, ACC_W), jnp.bfloat16),
            pltpu.VMEM((CHUNK, ACC_W), jnp.bfloat16),
            pltpu.VMEM((CHUNK, ACC_W), jnp.bfloat16),
            pltpu.VMEM((CHUNK, ACC_W), jnp.bfloat16),
            pltpu.VMEM((CHUNK, ACC_W), jnp.float32),
            pltpu.VMEM((CHUNK, ACC_W), jnp.float32),
            pltpu.VMEM_SHARED((NP, ACC_W), jnp.float32),
            pltpu.SemaphoreType.DMA,
            pltpu.SemaphoreType.DMA,
            pltpu.SemaphoreType.DMA,
            pltpu.SemaphoreType.DMA,
            pltpu.SemaphoreType.DMA,
            pltpu.SemaphoreType.DMA,
        ],
        compiler_params=_sc_compiler_params(),
    )
    def k(src_hbm, dst_hbm, h_hbm, dp_hbm, out_hbm,
          sidx, didx, dbuf0, dbuf1, dbuf2, dbuf3, hbuf0, hbuf1, hbuf2, hbuf3,
          obuf0, obuf1, acc, gsem0, gsem1, gsem2, gsem3, ssem0, ssem1):
        cid = lax.axis_index("c")
        sid = lax.axis_index("s")
        wid = sid * 2 + cid
        c0 = wid * NCH_W

        dbuf = (dbuf0, dbuf1, dbuf2, dbuf3)
        hbuf = (hbuf0, hbuf1, hbuf2, hbuf3)
        obuf = (obuf0, obuf1)
        gsem = (gsem0, gsem1, gsem2, gsem3)
        ssem = (ssem0, ssem1)

        zv = jnp.zeros((16,), jnp.float32)

        @pl.loop(0, CHUNK)
        def _(r):
            for c in range(ACC_W // 16):
                obuf0[r, pl.ds(c * 16, 16)] = zv

        for b in range(ROWS_PER_TILE // CHUNK):
            pltpu.sync_copy(obuf0, acc.at[pl.ds(sid * ROWS_PER_TILE + b * CHUNK, CHUNK)])
        plsc.subcore_barrier()

        def refill_idx(blk):
            pltpu.sync_copy(src_hbm.at[pl.ds(c0 + blk * IBLK, IBLK)], sidx)
            pltpu.sync_copy(dst_hbm.at[pl.ds(c0 + blk * IBLK, IBLK)], didx)

        def issue(row, b):
            pltpu.async_copy(dp_hbm.at[didx.at[row]], dbuf[b], gsem[b])
            pltpu.async_copy(h_hbm.at[sidx.at[row]], hbuf[b], gsem[b])

        def drain(b):
            pltpu.make_async_copy(dp_hbm.at[didx.at[0]], dbuf[b], gsem[b]).wait()
            pltpu.make_async_copy(h_hbm.at[sidx.at[0]], hbuf[b], gsem[b]).wait()

        iota16 = lax.iota(jnp.int32, 16)
        shift8 = jnp.minimum(iota16 + 8, 15)
        head_idx = [jnp.full((16,), hh, jnp.int32) for hh in range(HEADS)]

        def compute(g, p, it, row):
            drain(g)
            ob = obuf[p]
            hb = hbuf[g]

            @pl.when(row >= 2)
            def _():
                pltpu.make_async_copy(ob, acc.at[didx.at[0]], ssem[p]).wait()

            @plsc.parallel_loop(0, CHUNK, unroll=4)
            def _(e):
                r2 = hb[e, pl.ds(64, 32)]
                h4, sreg = plsc.unpack(
                    r2, format=plsc.PackFormat.INTERLEAVED,
                    preferred_element_type=jnp.float32)
                dreg = dbuf[g][e]
                u = sreg + dreg
                l = jnp.maximum(u, 0.2 * u)
                ub = _dg(dreg, shift8)
                t = jnp.exp(l - ub)
                ob[e, pl.ds(80, 16)] = t
                r0 = hb[e, pl.ds(0, 32)]
                h0, h1 = plsc.unpack(
                    r0, format=plsc.PackFormat.INTERLEAVED,
                    preferred_element_type=jnp.float32)
                r1 = hb[e, pl.ds(32, 32)]
                h2, h3 = plsc.unpack(
                    r1, format=plsc.PackFormat.INTERLEAVED,
                    preferred_element_type=jnp.float32)
                hs = (h0, h1, h2, h3, h4)
                for hh in range(HEADS):
                    cf = _dg(t, head_idx[hh])
                    ob[e, pl.ds(hh * 16, 16)] = hs[hh] * cf

            pltpu.async_copy(ob, acc.at[didx.at[row]], ssem[p], add=True)

        refill_idx(0)
        issue(0, 0)
        issue(1, 1)

        @pl.loop(0, NBLK)
        def _(bb):
            @pl.loop(0, IBLK // 4)
            def _(cc):
                for q in range(4):
                    j = cc * 4 + q
                    it = bb * IBLK + j
                    p = q % 2

                    if q < 2:
                        @pl.when(c0 + it + 2 < NCHUNKS)
                        def _():
                            issue(j + 2, (q + 2) % 4)
                    else:
                        @pl.when((cc < IBLK // 4 - 1) & (c0 + it + 2 < NCHUNKS))
                        def _():
                            issue(j + 2, (q + 2) % 4)

                    @pl.when(c0 + it < NCHUNKS)
                    def _():
                        compute(q, p, it, j)

                    if q == 3:
                        @pl.when((cc == IBLK // 4 - 1) & (bb + 1 < NBLK))
                        def _():
                            @pl.when(c0 + it - 1 < NCHUNKS)
                            def _():
                                pltpu.make_async_copy(
                                    obuf[0], acc.at[didx.at[0]], ssem[0]).wait()

                            @pl.when(c0 + it < NCHUNKS)
                            def _():
                                pltpu.make_async_copy(
                                    obuf[1], acc.at[didx.at[0]], ssem[1]).wait()

                            refill_idx(bb + 1)

                            @pl.when(c0 + it + 1 < NCHUNKS)
                            def _():
                                issue(0, 0)

                            @pl.when(c0 + it + 2 < NCHUNKS)
                            def _():
                                issue(1, 1)

        pltpu.make_async_copy(obuf[0], acc.at[didx.at[0]], ssem[0]).wait()
        pltpu.make_async_copy(obuf[1], acc.at[didx.at[0]], ssem[1]).wait()
        plsc.subcore_barrier()
        pltpu.sync_copy(acc.at[pl.ds(sid * ROWS_PER_TILE, ROWS_PER_TILE)],
                        out_hbm.at[cid, pl.ds(sid * ROWS_PER_TILE, ROWS_PER_TILE)])

    return k(src2d, dst2d, h, dp)


def _head_sum_mat(att_flat):
    r = lax.broadcasted_iota(jnp.int32, (80, HEADS), 0)
    c = lax.broadcasted_iota(jnp.int32, (80, HEADS), 1)
    sel = (r // 16 == c).astype(jnp.float32)
    return att_flat[:, None] * sel


def _expand_mat():
    r = lax.broadcasted_iota(jnp.int32, (HEADS, 80), 0)
    c = lax.broadcasted_iota(jnp.int32, (HEADS, 80), 1)
    return (c // 16 == r).astype(jnp.float32)


def _perm_mat():
    lrow = lax.broadcasted_iota(jnp.int32, (96, 96), 0)
    s = lax.broadcasted_iota(jnp.int32, (96, 96), 1)
    q = s // 32
    r = s % 32
    l_of_s = 32 * q + r // 2 + 16 * (r % 2)
    return (lrow == l_of_s).astype(jnp.float32)


def _attn_packs(hmat, att_s_flat, att_d_flat):
    a_s = jnp.dot(hmat, _head_sum_mat(att_s_flat),
                  preferred_element_type=jnp.float32)
    a_d = jnp.dot(hmat, _head_sum_mat(att_d_flat),
                  preferred_element_type=jnp.float32)
    gmax = jnp.max(a_s, axis=0, keepdims=True)
    v = gmax + a_d
    ub = jnp.maximum(v, 0.2 * v)
    z = jnp.zeros((hmat.shape[0], 3), jnp.float32)
    sp = jnp.concatenate([a_s, z, a_s, z], axis=1)
    dp = jnp.concatenate([a_d, z, ub, z], axis=1)
    return sp, dp


def _pre1_body(x_ref, wemb_ref, bemb_ref, w1_ref, as1_ref, ad1_ref,
               emb_ref, h_ref, dp_ref):
    emb = jnp.dot(x_ref[...], wemb_ref[...],
                  preferred_element_type=jnp.float32) + bemb_ref[...]
    emb_ref[...] = emb
    h = jnp.dot(emb, w1_ref[...], preferred_element_type=jnp.float32)
    sp, dp = _attn_packs(h, as1_ref[...][0], ad1_ref[...][0])
    h96 = jnp.concatenate([h, sp], axis=1)
    h_ref[...] = jnp.dot(h96, _perm_mat(),
                         preferred_element_type=jnp.float32).astype(jnp.bfloat16)
    dp_ref[...] = dp


def _mid_body(acc_ref, b1_ref, w2_ref, as2_ref, ad2_ref,
              h_ref, dp_ref):
    s = acc_ref[0] + acc_ref[1]
    msg = s[:, :80]
    den = jnp.dot(s[:, 80:85], _expand_mat(),
                  preferred_element_type=jnp.float32)
    o = msg / (den + 1e-16) + b1_ref[...]
    x2 = jnp.where(o > 0, o, jnp.exp(jnp.minimum(o, 0.0)) - 1.0)
    h = jnp.dot(x2, w2_ref[...], preferred_element_type=jnp.float32)
    sp, dp = _attn_packs(h, as2_ref[...][0], ad2_ref[...][0])
    h96 = jnp.concatenate([h, sp], axis=1)
    h_ref[...] = jnp.dot(h96, _perm_mat(),
                         preferred_element_type=jnp.float32).astype(jnp.bfloat16)
    dp_ref[...] = dp


def _post_body(acc_ref, b2_ref, out_ref):
    s = acc_ref[0] + acc_ref[1]
    msg = s[:, :80]
    den = jnp.dot(s[:, 80:85], _expand_mat(),
                  preferred_element_type=jnp.float32)
    o = msg / (den + 1e-16)
    r = lax.broadcasted_iota(jnp.int32, (80, OUT_CH), 0)
    c = lax.broadcasted_iota(jnp.int32, (80, OUT_CH), 1)
    mh = (r % 16 == c).astype(jnp.float32) / HEADS
    om = jnp.dot(o, mh, preferred_element_type=jnp.float32) + b2_ref[...]
    m = jnp.max(om, axis=1, keepdims=True)
    z = om - m
    lse = jnp.log(jnp.sum(jnp.exp(z), axis=1, keepdims=True))
    out_ref[...] = z - lse


def kernel(x, edge_index, W_emb, b_emb, W1, att_src1, att_dst1, b1,
           W2, att_src2, att_dst2, b2):
    pad_e = ECHUNKS_PAD * CHUNK - E
    src = jnp.concatenate([edge_index[0], jnp.zeros((pad_e,), jnp.int32)])
    src2d = src.reshape(ECHUNKS_PAD, CHUNK)
    dst = jnp.concatenate([edge_index[1], jnp.zeros((pad_e,), jnp.int32)])
    dst2d = dst.reshape(ECHUNKS_PAD, CHUNK)
    as1 = att_src1.reshape(1, HEADS * HID)
    ad1 = att_dst1.reshape(1, HEADS * HID)
    as2 = att_src2.reshape(1, HEADS * OUT_CH)
    ad2 = att_dst2.reshape(1, HEADS * OUT_CH)
    b_emb2 = b_emb.reshape(1, HID)
    b1r = b1.reshape(1, HEADS * HID)
    b2r = b2.reshape(1, OUT_CH)
    x_pad = jnp.concatenate(
        [x, jnp.zeros((NP - N, IN_CH), jnp.float32)], axis=0)

    emb, h1, dp1 = pl.pallas_call(
        _pre1_body,
        out_shape=[
            jax.ShapeDtypeStruct((NP, HID), jnp.float32),
            jax.ShapeDtypeStruct((NP, 96), jnp.bfloat16),
            jax.ShapeDtypeStruct((NP, 16), jnp.float32),
        ],
    )(x_pad, W_emb, b_emb2, W1, as1, ad1)

    acc1 = _edge_pass(src2d, dst2d, h1, dp1)

    h2, dp2 = pl.pallas_call(
        _mid_body,
        out_shape=[
            jax.ShapeDtypeStruct((NP, 96), jnp.bfloat16),
            jax.ShapeDtypeStruct((NP, 16), jnp.float32),
        ],
    )(acc1, b1r, W2, as2, ad2)

    acc2 = _edge_pass(src2d, dst2d, h2, dp2)

    out = pl.pallas_call(
        _post_body,
        out_shape=jax.ShapeDtypeStruct((NP, OUT_CH), jnp.float32),
    )(acc2, b2r)

    return (emb[:N], out[:N])
